# Initial kernel scaffold; baseline (speedup 1.0000x reference)
import jax
import jax.numpy as jnp
from jax.experimental import pallas as pl


def kernel(graph_nodes, graph_edge_links, agent_locations, mask, W0, b0, g0, be0, W1, b1, g1, be1, W2, b2, g2, be2, Wc1, bc1, gbn, bbn, Wc2, bc2):
    raise NotImplementedError("write your pallas kernel here")


# speedup vs baseline: 11.9602x; 11.9602x over previous
import functools

import jax
import jax.numpy as jnp
from jax import lax
from jax.experimental import pallas as pl
from jax.experimental.pallas import tpu as pltpu
from jax.experimental.pallas import tpu_sc as plsc

B = 4
NN = 10000
E = 150000
D = 128
HID = 128
BN = B * NN

NC = 2
NS = 16

K = 128
NITER = 74
T = NITER * K
EPAD = NS * T
ACCROWS = 10240
G = 8


def _sc_segment_sum(x, src3, dst3):
    mesh = plsc.VectorSubcoreMesh(
        core_axis_name="c", subcore_axis_name="s", num_cores=NC,
        num_subcores=NS)

    @functools.partial(
        pl.kernel,
        mesh=mesh,
        out_type=jax.ShapeDtypeStruct((BN, D), jnp.float32),
        scratch_types=[
            pltpu.VMEM((G, K), jnp.int32),
            pltpu.VMEM((G, K), jnp.int32),
            pltpu.VMEM((K,), jnp.int32),
            pltpu.VMEM((2, K, D), jnp.float32),
            pltpu.VMEM_SHARED((ACCROWS, D), jnp.float32),
            pltpu.SemaphoreType.DMA,
            pltpu.SemaphoreType.DMA,
        ],
    )
    def seg(x_hbm, src_hbm, dst_hbm, out_hbm, srcb, dstb, didx, gbuf,
            acc, sem_a, sem_b):
        c = lax.axis_index("c")
        s = lax.axis_index("s")
        zero16 = jnp.zeros((16,), jnp.float32)
        sems = (sem_a, sem_b)

        def zrow(r, carry):
            for j in range(D // 16):
                gbuf[0, r, pl.ds(j * 16, 16)] = zero16
            return carry

        def do_group(g, gsz):
            off = pl.multiple_of(g * G, G)
            pltpu.sync_copy(src_hbm.at[b, s, pl.ds(off, gsz)],
                            srcb.at[pl.ds(0, gsz)])
            pltpu.sync_copy(dst_hbm.at[b, s, pl.ds(off, gsz)],
                            dstb.at[pl.ds(0, gsz)])
            for r in range(gsz):
                for j in range(K // 16):
                    srcb[r, pl.ds(j * 16, 16)] = (
                        srcb[r, pl.ds(j * 16, 16)] + base_x)
            pltpu.async_copy(x_hbm.at[srcb.at[0]], gbuf.at[0], sems[0])
            for r in range(gsz):
                if r + 1 < gsz:
                    pltpu.async_copy(x_hbm.at[srcb.at[r + 1]],
                                     gbuf.at[(r + 1) % 2], sems[(r + 1) % 2])
                pltpu.make_async_copy(x_hbm.at[srcb.at[r]], gbuf.at[r % 2],
                                      sems[r % 2]).wait()
                for j in range(K // 16):
                    didx[pl.ds(j * 16, 16)] = dstb[r, pl.ds(j * 16, 16)]
                pltpu.sync_copy(gbuf.at[r % 2], acc.at[didx], add=True)

        for bb in range(B // NC):
            b = bb * NC + c
            base_x = b * NN
            plsc.subcore_barrier()
            lax.fori_loop(0, K, zrow, 0)
            for z in range(ACCROWS // (NS * K)):
                pltpu.sync_copy(gbuf.at[0],
                                acc.at[pl.ds(s * (ACCROWS // NS) + z * K, K)])
            plsc.subcore_barrier()

            lax.fori_loop(0, NITER // G, lambda g, cr: (do_group(g, G), cr)[1],
                          0)
            if NITER % G:
                do_group(NITER // G, NITER % G)
            plsc.subcore_barrier()
            rows_out = 624
            pltpu.sync_copy(acc.at[pl.ds(s * rows_out, rows_out)],
                            out_hbm.at[pl.ds(base_x + s * rows_out, rows_out)])

            @pl.when(s == NS - 1)
            def _():
                pltpu.sync_copy(acc.at[pl.ds(NS * rows_out, NN - NS * rows_out)],
                                out_hbm.at[pl.ds(base_x + NS * rows_out,
                                                 NN - NS * rows_out)])

    return seg(x, src3, dst3)


def _dense_layer(agg, W, bvec, g, be):
    R = 2000

    def body(a_ref, w_ref, b_ref, g_ref, be_ref, o_ref):
        h = jnp.dot(a_ref[...].astype(jnp.bfloat16),
                    w_ref[...].astype(jnp.bfloat16),
                    preferred_element_type=jnp.float32) + b_ref[...]
        mu = jnp.mean(h, axis=1, keepdims=True)
        xc = h - mu
        var = jnp.mean(xc * xc, axis=1, keepdims=True)
        r0 = lax.rsqrt(var + 1e-5)
        inv = r0 * (1.5 - 0.5 * (var + 1e-5) * r0 * r0)
        hn = xc * inv * g_ref[...] + be_ref[...]
        o_ref[...] = jnp.maximum(hn, 0.0)

    return pl.pallas_call(
        body,
        grid=(BN // R,),
        in_specs=[
            pl.BlockSpec((R, D), lambda i: (i, 0)),
            pl.BlockSpec((D, HID), lambda i: (0, 0)),
            pl.BlockSpec((1, HID), lambda i: (0, 0)),
            pl.BlockSpec((1, HID), lambda i: (0, 0)),
            pl.BlockSpec((1, HID), lambda i: (0, 0)),
        ],
        out_specs=pl.BlockSpec((R, HID), lambda i: (i, 0)),
        out_shape=jax.ShapeDtypeStruct((BN, HID), jnp.float32),
    )(agg, W, bvec.reshape(1, HID), g.reshape(1, HID), be.reshape(1, HID))


def _head_stage1(x0, x1, x2, x3, Wc1, bc1):
    R = 2000
    C = 2 * HID

    def body(x0r, x1r, x2r, x3r, wr, br, h_ref, sum_ref, ssq_ref):
        w = wr[...].astype(jnp.bfloat16)
        h = (jnp.dot(x0r[...].astype(jnp.bfloat16), w[0:D],
                     preferred_element_type=jnp.float32)
             + jnp.dot(x1r[...].astype(jnp.bfloat16), w[D:2 * D],
                       preferred_element_type=jnp.float32)
             + jnp.dot(x2r[...].astype(jnp.bfloat16), w[2 * D:3 * D],
                       preferred_element_type=jnp.float32)
             + jnp.dot(x3r[...].astype(jnp.bfloat16), w[3 * D:4 * D],
                       preferred_element_type=jnp.float32)
             + br[...])
        h_ref[...] = h

        @pl.when(pl.program_id(0) == 0)
        def _():
            sum_ref[...] = jnp.zeros_like(sum_ref)
            ssq_ref[...] = jnp.zeros_like(ssq_ref)

        sum_ref[...] += jnp.sum(h, axis=0, keepdims=True)
        ssq_ref[...] += jnp.sum(h * h, axis=0, keepdims=True)

    return pl.pallas_call(
        body,
        grid=(BN // R,),
        in_specs=[
            pl.BlockSpec((R, D), lambda i: (i, 0)),
            pl.BlockSpec((R, D), lambda i: (i, 0)),
            pl.BlockSpec((R, D), lambda i: (i, 0)),
            pl.BlockSpec((R, D), lambda i: (i, 0)),
            pl.BlockSpec((4 * D, C), lambda i: (0, 0)),
            pl.BlockSpec((1, C), lambda i: (0, 0)),
        ],
        out_specs=[
            pl.BlockSpec((R, C), lambda i: (i, 0)),
            pl.BlockSpec((1, C), lambda i: (0, 0)),
            pl.BlockSpec((1, C), lambda i: (0, 0)),
        ],
        out_shape=[
            jax.ShapeDtypeStruct((BN, C), jnp.float32),
            jax.ShapeDtypeStruct((1, C), jnp.float32),
            jax.ShapeDtypeStruct((1, C), jnp.float32),
        ],
    )(x0, x1, x2, x3, Wc1, bc1.reshape(1, C))


def _head_stage2(h, sums, ssq, gbn, bbn, Wc2, bc2):
    R = 2000
    C = 2 * HID
    blocks_per_batch = NN // R

    def body(h_ref, sum_ref, ssq_ref, g_ref, b_ref, w2_ref, b2_ref, o_ref):
        j = pl.program_id(0)
        mu = sum_ref[...] * (1.0 / BN)
        var = ssq_ref[...] * (1.0 / BN) - mu * mu
        r0 = lax.rsqrt(var + 1e-5)
        inv = r0 * (1.5 - 0.5 * (var + 1e-5) * r0 * r0)
        scale = g_ref[...] * inv
        shift = b_ref[...] - mu * scale
        r = jnp.maximum(h_ref[...] * scale + shift, 0.0)
        r16 = r.astype(jnp.bfloat16).astype(jnp.float32)
        w16 = w2_ref[...].astype(jnp.bfloat16).astype(jnp.float32)
        colsum = jnp.sum(r16, axis=0, keepdims=True)
        val = jnp.sum(colsum * w16)

        @pl.when(j == 0)
        def _():
            o_ref[...] = jnp.zeros_like(o_ref)

        bi = j // blocks_per_batch
        o_ref[pl.ds(bi, 1), :] += jnp.full((1, D), val, jnp.float32)

        @pl.when(j == BN // R - 1)
        def _():
            o_ref[...] = o_ref[...] * (1.0 / NN) + b2_ref[...]

    out = pl.pallas_call(
        body,
        grid=(BN // R,),
        in_specs=[
            pl.BlockSpec((R, C), lambda i: (i, 0)),
            pl.BlockSpec((1, C), lambda i: (0, 0)),
            pl.BlockSpec((1, C), lambda i: (0, 0)),
            pl.BlockSpec((1, C), lambda i: (0, 0)),
            pl.BlockSpec((1, C), lambda i: (0, 0)),
            pl.BlockSpec((1, C), lambda i: (0, 0)),
            pl.BlockSpec((1, 1), lambda i: (0, 0)),
        ],
        out_specs=pl.BlockSpec((B, D), lambda i: (0, 0)),
        out_shape=jax.ShapeDtypeStruct((B, D), jnp.float32),
    )(h, sums, ssq, gbn.reshape(1, C), bbn.reshape(1, C),
      Wc2.reshape(1, C), bc2.reshape(1, 1))
    return out[:, :1]


def kernel(graph_nodes, graph_edge_links, agent_locations, mask, W0, b0, g0,
           be0, W1, b1, g1, be1, W2, b2, g2, be2, Wc1, bc1, gbn, bbn, Wc2,
           bc2):
    del agent_locations, mask
    x0 = graph_nodes.reshape(BN, D)

    src = graph_edge_links[:, 0, :]
    dst = graph_edge_links[:, 1, :]
    npad = EPAD - E
    padidx = jnp.arange(npad, dtype=jnp.int32)
    pad_src = jnp.broadcast_to(padidx % NN, (B, npad))
    pad_dst = jnp.broadcast_to(NN + padidx % (ACCROWS - NN), (B, npad))
    src3 = jnp.concatenate([src, pad_src], axis=1).reshape(B, NS, NITER, K)
    dst3 = jnp.concatenate([dst, pad_dst], axis=1).reshape(B, NS, NITER, K)

    params = [(W0, b0, g0, be0), (W1, b1, g1, be1), (W2, b2, g2, be2)]
    xs = [x0]
    x = x0
    for (W, bvec, g, be) in params:
        agg = _sc_segment_sum(x, src3, dst3)
        x = _dense_layer(agg, W, bvec, g, be)
        xs.append(x)

    h, sums, ssq = _head_stage1(xs[0], xs[1], xs[2], xs[3], Wc1, bc1)
    return _head_stage2(h, sums, ssq, gbn, bbn, Wc2, bc2)


# speedup vs baseline: 14.1663x; 1.1845x over previous
import functools

import jax
import jax.numpy as jnp
from jax import lax
from jax.experimental import pallas as pl
from jax.experimental.pallas import tpu as pltpu
from jax.experimental.pallas import tpu_sc as plsc

B = 4
NN = 10000
E = 150000
D = 128
HID = 128
BN = B * NN

NC = 2
NS = 16

K = 128
NITER = 74
T = NITER * K
EPAD = NS * T
ACCROWS = 10240
G = 8


def _sc_segment_sum(x, src3, dst3):
    mesh = plsc.VectorSubcoreMesh(
        core_axis_name="c", subcore_axis_name="s", num_cores=NC,
        num_subcores=NS)

    ngrp = (NITER + G - 1) // G
    sizes = [G] * (NITER // G) + ([NITER % G] if NITER % G else [])

    @functools.partial(
        pl.kernel,
        mesh=mesh,
        out_type=jax.ShapeDtypeStruct((BN, D), jnp.float32),
        scratch_types=[
            pltpu.VMEM((2, G, K), jnp.int32),
            pltpu.VMEM((2, G, K), jnp.int32),
            pltpu.VMEM((2, K, D), jnp.float32),
            pltpu.VMEM_SHARED((ACCROWS, D), jnp.float32),
            pltpu.SemaphoreType.DMA,
            pltpu.SemaphoreType.DMA,
            pltpu.SemaphoreType.DMA,
        ],
    )
    def seg(x_hbm, src_hbm, dst_hbm, out_hbm, srcb, dstb, gbuf,
            acc, sem_a, sem_b, sem_i):
        c = lax.axis_index("c")
        s = lax.axis_index("s")
        zero16 = jnp.zeros((16,), jnp.float32)
        sems = (sem_a, sem_b)

        def zrow(r, carry):
            for j in range(D // 16):
                gbuf[0, r, pl.ds(j * 16, 16)] = zero16
            return carry

        def idx_copies(b, g):
            gi = g % 2
            gsz = sizes[g]
            return (
                pltpu.make_async_copy(
                    src_hbm.at[b, s, pl.ds(g * G, gsz)],
                    srcb.at[gi, pl.ds(0, gsz)], sem_i),
                pltpu.make_async_copy(
                    dst_hbm.at[b, s, pl.ds(g * G, gsz)],
                    dstb.at[gi, pl.ds(0, gsz)], sem_i),
            )

        for bb in range(B // NC):
            b = bb * NC + c
            base_x = b * NN
            plsc.subcore_barrier()
            for cp in idx_copies(b, 0):
                cp.start()
            lax.fori_loop(0, K, zrow, 0)
            for z in range(ACCROWS // (NS * K)):
                pltpu.sync_copy(gbuf.at[0],
                                acc.at[pl.ds(s * (ACCROWS // NS) + z * K, K)])
            plsc.subcore_barrier()

            for cp in idx_copies(b, 0):
                cp.wait()
            pltpu.async_copy(x_hbm.at[srcb.at[0, 0]], gbuf.at[0], sems[0])
            q = 0
            for g in range(ngrp):
                gi = g % 2
                if g + 1 < ngrp:
                    for cp in idx_copies(b, g + 1):
                        cp.start()
                for r in range(sizes[g]):
                    if r + 1 < sizes[g]:
                        nxt = (g, r + 1)
                    elif g + 1 < ngrp:
                        nxt = (g + 1, 0)
                    else:
                        nxt = None
                    if nxt is not None:
                        if nxt[1] == 0:
                            for cp in idx_copies(b, nxt[0]):
                                cp.wait()
                        pltpu.async_copy(
                            x_hbm.at[srcb.at[nxt[0] % 2, nxt[1]]],
                            gbuf.at[(q + 1) % 2], sems[(q + 1) % 2])
                    pltpu.make_async_copy(x_hbm.at[srcb.at[gi, r]],
                                          gbuf.at[q % 2], sems[q % 2]).wait()
                    pltpu.sync_copy(gbuf.at[q % 2], acc.at[dstb.at[gi, r]],
                                    add=True)
                    q += 1
            plsc.subcore_barrier()
            rows_out = 624
            pltpu.sync_copy(acc.at[pl.ds(s * rows_out, rows_out)],
                            out_hbm.at[pl.ds(base_x + s * rows_out, rows_out)])

            @pl.when(s == NS - 1)
            def _():
                pltpu.sync_copy(acc.at[pl.ds(NS * rows_out, NN - NS * rows_out)],
                                out_hbm.at[pl.ds(base_x + NS * rows_out,
                                                 NN - NS * rows_out)])

    return seg(x, src3, dst3)


def _dense_layer(agg, W, bvec, g, be):
    R = 2000

    def body(a_ref, w_ref, b_ref, g_ref, be_ref, o_ref):
        h = jnp.dot(a_ref[...].astype(jnp.bfloat16),
                    w_ref[...].astype(jnp.bfloat16),
                    preferred_element_type=jnp.float32) + b_ref[...]
        mu = jnp.mean(h, axis=1, keepdims=True)
        xc = h - mu
        var = jnp.mean(xc * xc, axis=1, keepdims=True)
        r0 = lax.rsqrt(var + 1e-5)
        inv = r0 * (1.5 - 0.5 * (var + 1e-5) * r0 * r0)
        hn = xc * inv * g_ref[...] + be_ref[...]
        o_ref[...] = jnp.maximum(hn, 0.0)

    return pl.pallas_call(
        body,
        grid=(BN // R,),
        in_specs=[
            pl.BlockSpec((R, D), lambda i: (i, 0)),
            pl.BlockSpec((D, HID), lambda i: (0, 0)),
            pl.BlockSpec((1, HID), lambda i: (0, 0)),
            pl.BlockSpec((1, HID), lambda i: (0, 0)),
            pl.BlockSpec((1, HID), lambda i: (0, 0)),
        ],
        out_specs=pl.BlockSpec((R, HID), lambda i: (i, 0)),
        out_shape=jax.ShapeDtypeStruct((BN, HID), jnp.float32),
    )(agg, W, bvec.reshape(1, HID), g.reshape(1, HID), be.reshape(1, HID))


def _head_stage1(x0, x1, x2, x3, Wc1, bc1):
    R = 2000
    C = 2 * HID

    def body(x0r, x1r, x2r, x3r, wr, br, h_ref, sum_ref, ssq_ref):
        w = wr[...].astype(jnp.bfloat16)
        h = (jnp.dot(x0r[...].astype(jnp.bfloat16), w[0:D],
                     preferred_element_type=jnp.float32)
             + jnp.dot(x1r[...].astype(jnp.bfloat16), w[D:2 * D],
                       preferred_element_type=jnp.float32)
             + jnp.dot(x2r[...].astype(jnp.bfloat16), w[2 * D:3 * D],
                       preferred_element_type=jnp.float32)
             + jnp.dot(x3r[...].astype(jnp.bfloat16), w[3 * D:4 * D],
                       preferred_element_type=jnp.float32)
             + br[...])
        h_ref[...] = h

        @pl.when(pl.program_id(0) == 0)
        def _():
            sum_ref[...] = jnp.zeros_like(sum_ref)
            ssq_ref[...] = jnp.zeros_like(ssq_ref)

        sum_ref[...] += jnp.sum(h, axis=0, keepdims=True)
        ssq_ref[...] += jnp.sum(h * h, axis=0, keepdims=True)

    return pl.pallas_call(
        body,
        grid=(BN // R,),
        in_specs=[
            pl.BlockSpec((R, D), lambda i: (i, 0)),
            pl.BlockSpec((R, D), lambda i: (i, 0)),
            pl.BlockSpec((R, D), lambda i: (i, 0)),
            pl.BlockSpec((R, D), lambda i: (i, 0)),
            pl.BlockSpec((4 * D, C), lambda i: (0, 0)),
            pl.BlockSpec((1, C), lambda i: (0, 0)),
        ],
        out_specs=[
            pl.BlockSpec((R, C), lambda i: (i, 0)),
            pl.BlockSpec((1, C), lambda i: (0, 0)),
            pl.BlockSpec((1, C), lambda i: (0, 0)),
        ],
        out_shape=[
            jax.ShapeDtypeStruct((BN, C), jnp.float32),
            jax.ShapeDtypeStruct((1, C), jnp.float32),
            jax.ShapeDtypeStruct((1, C), jnp.float32),
        ],
    )(x0, x1, x2, x3, Wc1, bc1.reshape(1, C))


def _head_stage2(h, sums, ssq, gbn, bbn, Wc2, bc2):
    R = 2000
    C = 2 * HID
    blocks_per_batch = NN // R

    def body(h_ref, sum_ref, ssq_ref, g_ref, b_ref, w2_ref, b2_ref, o_ref):
        j = pl.program_id(0)
        mu = sum_ref[...] * (1.0 / BN)
        var = ssq_ref[...] * (1.0 / BN) - mu * mu
        r0 = lax.rsqrt(var + 1e-5)
        inv = r0 * (1.5 - 0.5 * (var + 1e-5) * r0 * r0)
        scale = g_ref[...] * inv
        shift = b_ref[...] - mu * scale
        r = jnp.maximum(h_ref[...] * scale + shift, 0.0)
        r16 = r.astype(jnp.bfloat16).astype(jnp.float32)
        w16 = w2_ref[...].astype(jnp.bfloat16).astype(jnp.float32)
        colsum = jnp.sum(r16, axis=0, keepdims=True)
        val = jnp.sum(colsum * w16)

        @pl.when(j == 0)
        def _():
            o_ref[...] = jnp.zeros_like(o_ref)

        bi = j // blocks_per_batch
        o_ref[pl.ds(bi, 1), :] += jnp.full((1, D), val, jnp.float32)

        @pl.when(j == BN // R - 1)
        def _():
            o_ref[...] = o_ref[...] * (1.0 / NN) + b2_ref[...]

    out = pl.pallas_call(
        body,
        grid=(BN // R,),
        in_specs=[
            pl.BlockSpec((R, C), lambda i: (i, 0)),
            pl.BlockSpec((1, C), lambda i: (0, 0)),
            pl.BlockSpec((1, C), lambda i: (0, 0)),
            pl.BlockSpec((1, C), lambda i: (0, 0)),
            pl.BlockSpec((1, C), lambda i: (0, 0)),
            pl.BlockSpec((1, C), lambda i: (0, 0)),
            pl.BlockSpec((1, 1), lambda i: (0, 0)),
        ],
        out_specs=pl.BlockSpec((B, D), lambda i: (0, 0)),
        out_shape=jax.ShapeDtypeStruct((B, D), jnp.float32),
    )(h, sums, ssq, gbn.reshape(1, C), bbn.reshape(1, C),
      Wc2.reshape(1, C), bc2.reshape(1, 1))
    return out[:, :1]


def kernel(graph_nodes, graph_edge_links, agent_locations, mask, W0, b0, g0,
           be0, W1, b1, g1, be1, W2, b2, g2, be2, Wc1, bc1, gbn, bbn, Wc2,
           bc2):
    del agent_locations, mask
    x0 = graph_nodes.reshape(BN, D)

    offs = (jnp.arange(B, dtype=jnp.int32) * NN)[:, None]
    src = graph_edge_links[:, 0, :] + offs
    dst = graph_edge_links[:, 1, :]
    npad = EPAD - E
    padidx = jnp.arange(npad, dtype=jnp.int32)
    pad_src = padidx % NN + offs
    pad_dst = jnp.broadcast_to(NN + padidx % (ACCROWS - NN), (B, npad))
    src3 = jnp.concatenate([src, pad_src], axis=1).reshape(B, NS, NITER, K)
    dst3 = jnp.concatenate([dst, pad_dst], axis=1).reshape(B, NS, NITER, K)

    params = [(W0, b0, g0, be0), (W1, b1, g1, be1), (W2, b2, g2, be2)]
    xs = [x0]
    x = x0
    for (W, bvec, g, be) in params:
        agg = _sc_segment_sum(x, src3, dst3)
        x = _dense_layer(agg, W, bvec, g, be)
        xs.append(x)

    h, sums, ssq = _head_stage1(xs[0], xs[1], xs[2], xs[3], Wc1, bc1)
    return _head_stage2(h, sums, ssq, gbn, bbn, Wc2, bc2)


# speedup vs baseline: 16.4294x; 1.1597x over previous
import functools

import jax
import jax.numpy as jnp
from jax import lax
from jax.experimental import pallas as pl
from jax.experimental.pallas import tpu as pltpu
from jax.experimental.pallas import tpu_sc as plsc

B = 4
NN = 10000
E = 150000
D = 128
HID = 128
BN = B * NN
PN = 2 * NN

NC = 2
NS = 16

K = 112
NITER = 84
T = NITER * K
EPAD = NS * T
ACCROWS = 10240
G = 8


def _sc_segment_sum(x, src3, dst3):
    mesh = plsc.VectorSubcoreMesh(
        core_axis_name="c", subcore_axis_name="s", num_cores=NC,
        num_subcores=NS)

    ngrp = (NITER + G - 1) // G
    sizes = [G] * (NITER // G) + ([NITER % G] if NITER % G else [])

    @functools.partial(
        pl.kernel,
        mesh=mesh,
        out_type=jax.ShapeDtypeStruct((2 * NN, D), jnp.float32),
        scratch_types=[
            pltpu.VMEM((2, G, K), jnp.int32),
            pltpu.VMEM((2, G, K), jnp.int32),
            pltpu.VMEM((3, K, D), jnp.float32),
            pltpu.VMEM((16, D), jnp.float32),
            pltpu.VMEM_SHARED((ACCROWS, D), jnp.float32),
            pltpu.SemaphoreType.DMA,
            pltpu.SemaphoreType.DMA,
            pltpu.SemaphoreType.DMA,
            pltpu.SemaphoreType.DMA,
            pltpu.SemaphoreType.DMA,
            pltpu.SemaphoreType.DMA,
            pltpu.SemaphoreType.DMA,
        ],
    )
    def seg(x_hbm, src_hbm, dst_hbm, out_hbm, srcb, dstb, gbuf, zbuf,
            acc, sem_a, sem_b, sem_c, sem_s0, sem_s1, sem_i, sem_z):
        c = lax.axis_index("c")
        s = lax.axis_index("s")
        zero16 = jnp.zeros((16,), jnp.float32)
        sems = (sem_a, sem_b, sem_c)
        ssems = (sem_s0, sem_s1)

        def zrow(r, carry):
            for j in range(D // 16):
                zbuf[r, pl.ds(j * 16, 16)] = zero16
            return carry

        def idx_copies(b, g):
            gi = g % 2
            gsz = sizes[g]
            return (
                pltpu.make_async_copy(
                    src_hbm.at[b, s, pl.ds(g * G, gsz)],
                    srcb.at[gi, pl.ds(0, gsz)], sem_i),
                pltpu.make_async_copy(
                    dst_hbm.at[b, s, pl.ds(g * G, gsz)],
                    dstb.at[gi, pl.ds(0, gsz)], sem_i),
            )

        if True:
            b = c
            base_x = c * NN
            plsc.subcore_barrier()
            for cp in idx_copies(b, 0):
                cp.start()
            lax.fori_loop(0, 16, zrow, 0)
            zds = []
            for zi in range(ACCROWS // NS // 16):
                zd = pltpu.make_async_copy(
                    zbuf, acc.at[pl.ds(s * (ACCROWS // NS) + zi * 16, 16)],
                    sem_z)
                zd.start()
                zds.append(zd)

            chunks = [(g, r) for g in range(ngrp) for r in range(sizes[g])]
            nq = len(chunks)
            for cp in idx_copies(b, 0):
                cp.wait()
            gd = [None] * nq
            sd = [None] * nq
            for q0 in range(2):
                g0, r0 = chunks[q0]
                gd[q0] = pltpu.make_async_copy(
                    x_hbm.at[srcb.at[g0 % 2, r0]], gbuf.at[q0], sems[q0])
                gd[q0].start()
            for zd in zds:
                zd.wait()
            plsc.subcore_barrier()
            for q, (g, r) in enumerate(chunks):
                if r == 0 and g + 1 < ngrp:
                    for cp in idx_copies(b, g + 1):
                        cp.start()
                if q + 2 < nq:
                    g2, r2 = chunks[q + 2]
                    if r2 == 0:
                        for cp in idx_copies(b, g2):
                            cp.wait()
                    if q >= 1:
                        sd[q - 1].wait()
                    gd[q + 2] = pltpu.make_async_copy(
                        x_hbm.at[srcb.at[g2 % 2, r2]],
                        gbuf.at[(q + 2) % 3], sems[(q + 2) % 3])
                    gd[q + 2].start()
                elif q >= 1:
                    sd[q - 1].wait()
                gd[q].wait()
                sd[q] = pltpu.make_async_copy(
                    gbuf.at[q % 3], acc.at[dstb.at[g % 2, r]], ssems[q % 2])
                sd[q].start(add=True)
            sd[nq - 1].wait()
            plsc.subcore_barrier()
            rows_out = 624
            pltpu.sync_copy(acc.at[pl.ds(s * rows_out, rows_out)],
                            out_hbm.at[pl.ds(base_x + s * rows_out, rows_out)])

            @pl.when(s == NS - 1)
            def _():
                pltpu.sync_copy(acc.at[pl.ds(NS * rows_out, NN - NS * rows_out)],
                                out_hbm.at[pl.ds(base_x + NS * rows_out,
                                                 NN - NS * rows_out)])

    return seg(x, src3, dst3)


def _dense_layer(agg, W, bvec, g, be):
    R = 2000

    def body(a_ref, w_ref, b_ref, g_ref, be_ref, o_ref):
        h = jnp.dot(a_ref[...].astype(jnp.bfloat16),
                    w_ref[...].astype(jnp.bfloat16),
                    preferred_element_type=jnp.float32) + b_ref[...]
        mu = jnp.mean(h, axis=1, keepdims=True)
        xc = h - mu
        var = jnp.mean(xc * xc, axis=1, keepdims=True)
        r0 = lax.rsqrt(var + 1e-5)
        inv = r0 * (1.5 - 0.5 * (var + 1e-5) * r0 * r0)
        hn = xc * inv * g_ref[...] + be_ref[...]
        o_ref[...] = jnp.maximum(hn, 0.0)

    return pl.pallas_call(
        body,
        grid=(PN // R,),
        in_specs=[
            pl.BlockSpec((R, D), lambda i: (i, 0)),
            pl.BlockSpec((D, HID), lambda i: (0, 0)),
            pl.BlockSpec((1, HID), lambda i: (0, 0)),
            pl.BlockSpec((1, HID), lambda i: (0, 0)),
            pl.BlockSpec((1, HID), lambda i: (0, 0)),
        ],
        out_specs=pl.BlockSpec((R, HID), lambda i: (i, 0)),
        out_shape=jax.ShapeDtypeStruct((PN, HID), jnp.float32),
    )(agg, W, bvec.reshape(1, HID), g.reshape(1, HID), be.reshape(1, HID))


def _head_stage1(x0, x1, x2, x3, Wc1, bc1):
    R = 2000
    C = 2 * HID

    def body(x0r, x1r, x2r, x3r, wr, br, h_ref, sum_ref, ssq_ref):
        w = wr[...].astype(jnp.bfloat16)
        h = (jnp.dot(x0r[...].astype(jnp.bfloat16), w[0:D],
                     preferred_element_type=jnp.float32)
             + jnp.dot(x1r[...].astype(jnp.bfloat16), w[D:2 * D],
                       preferred_element_type=jnp.float32)
             + jnp.dot(x2r[...].astype(jnp.bfloat16), w[2 * D:3 * D],
                       preferred_element_type=jnp.float32)
             + jnp.dot(x3r[...].astype(jnp.bfloat16), w[3 * D:4 * D],
                       preferred_element_type=jnp.float32)
             + br[...])
        h_ref[...] = h

        @pl.when(pl.program_id(0) == 0)
        def _():
            sum_ref[...] = jnp.zeros_like(sum_ref)
            ssq_ref[...] = jnp.zeros_like(ssq_ref)

        sum_ref[...] += jnp.sum(h, axis=0, keepdims=True)
        ssq_ref[...] += jnp.sum(h * h, axis=0, keepdims=True)

    return pl.pallas_call(
        body,
        grid=(PN // R,),
        in_specs=[
            pl.BlockSpec((R, D), lambda i: (i, 0)),
            pl.BlockSpec((R, D), lambda i: (i, 0)),
            pl.BlockSpec((R, D), lambda i: (i, 0)),
            pl.BlockSpec((R, D), lambda i: (i, 0)),
            pl.BlockSpec((4 * D, C), lambda i: (0, 0)),
            pl.BlockSpec((1, C), lambda i: (0, 0)),
        ],
        out_specs=[
            pl.BlockSpec((R, C), lambda i: (i, 0)),
            pl.BlockSpec((1, C), lambda i: (0, 0)),
            pl.BlockSpec((1, C), lambda i: (0, 0)),
        ],
        out_shape=[
            jax.ShapeDtypeStruct((PN, C), jnp.float32),
            jax.ShapeDtypeStruct((1, C), jnp.float32),
            jax.ShapeDtypeStruct((1, C), jnp.float32),
        ],
    )(x0, x1, x2, x3, Wc1, bc1.reshape(1, C))


def _head_stage2(h, sums_a, ssq_a, sums_b, ssq_b, gbn, bbn, Wc2, bc2):
    R = 2000
    C = 2 * HID
    blocks_per_batch = NN // R

    def body(h_ref, suma_ref, ssqa_ref, sumb_ref, ssqb_ref, g_ref, b_ref,
             w2_ref, b2_ref, o_ref):
        j = pl.program_id(0)
        mu = (suma_ref[...] + sumb_ref[...]) * (1.0 / BN)
        var = (ssqa_ref[...] + ssqb_ref[...]) * (1.0 / BN) - mu * mu
        r0 = lax.rsqrt(var + 1e-5)
        inv = r0 * (1.5 - 0.5 * (var + 1e-5) * r0 * r0)
        scale = g_ref[...] * inv
        shift = b_ref[...] - mu * scale
        r = jnp.maximum(h_ref[...] * scale + shift, 0.0)
        r16 = r.astype(jnp.bfloat16).astype(jnp.float32)
        w16 = w2_ref[...].astype(jnp.bfloat16).astype(jnp.float32)
        colsum = jnp.sum(r16, axis=0, keepdims=True)
        val = jnp.sum(colsum * w16)

        @pl.when(j == 0)
        def _():
            o_ref[...] = jnp.zeros_like(o_ref)

        bi = j // blocks_per_batch
        o_ref[pl.ds(bi, 1), :] += jnp.full((1, D), val, jnp.float32)

        @pl.when(j == PN // R - 1)
        def _():
            o_ref[...] = o_ref[...] * (1.0 / NN) + b2_ref[...]

    out = pl.pallas_call(
        body,
        grid=(PN // R,),
        in_specs=[
            pl.BlockSpec((R, C), lambda i: (i, 0)),
            pl.BlockSpec((1, C), lambda i: (0, 0)),
            pl.BlockSpec((1, C), lambda i: (0, 0)),
            pl.BlockSpec((1, C), lambda i: (0, 0)),
            pl.BlockSpec((1, C), lambda i: (0, 0)),
            pl.BlockSpec((1, C), lambda i: (0, 0)),
            pl.BlockSpec((1, C), lambda i: (0, 0)),
            pl.BlockSpec((1, C), lambda i: (0, 0)),
            pl.BlockSpec((1, 1), lambda i: (0, 0)),
        ],
        out_specs=pl.BlockSpec((2, D), lambda i: (0, 0)),
        out_shape=jax.ShapeDtypeStruct((2, D), jnp.float32),
    )(h, sums_a, ssq_a, sums_b, ssq_b, gbn.reshape(1, C), bbn.reshape(1, C),
      Wc2.reshape(1, C), bc2.reshape(1, 1))
    return out[:, :1]


def kernel(graph_nodes, graph_edge_links, agent_locations, mask, W0, b0, g0,
           be0, W1, b1, g1, be1, W2, b2, g2, be2, Wc1, bc1, gbn, bbn, Wc2,
           bc2):
    del agent_locations, mask

    offs = ((jnp.arange(B, dtype=jnp.int32) % 2) * NN)[:, None]
    src = graph_edge_links[:, 0, :] + offs
    dst = graph_edge_links[:, 1, :]
    npad = EPAD - E
    padidx = jnp.arange(npad, dtype=jnp.int32)
    pad_src = padidx % NN + offs
    pad_dst = jnp.broadcast_to(NN + padidx % (ACCROWS - NN), (B, npad))
    src3 = jnp.concatenate([src, pad_src], axis=1).reshape(B, NS, NITER, K)
    dst3 = jnp.concatenate([dst, pad_dst], axis=1).reshape(B, NS, NITER, K)

    params = [(W0, b0, g0, be0), (W1, b1, g1, be1), (W2, b2, g2, be2)]
    xs_p = []
    for p in range(2):
        x = graph_nodes[2 * p:2 * p + 2].reshape(PN, D)
        xs = [x]
        for (W, bvec, g, be) in params:
            agg = _sc_segment_sum(x, src3[2 * p:2 * p + 2],
                                  dst3[2 * p:2 * p + 2])
            x = _dense_layer(agg, W, bvec, g, be)
            xs.append(x)
        xs_p.append(xs)

    h0, sums0, ssq0 = _head_stage1(*xs_p[0], Wc1, bc1)
    h1, sums1, ssq1 = _head_stage1(*xs_p[1], Wc1, bc1)
    o0 = _head_stage2(h0, sums0, ssq0, sums1, ssq1, gbn, bbn, Wc2, bc2)
    o1 = _head_stage2(h1, sums1, ssq1, sums0, ssq0, gbn, bbn, Wc2, bc2)
    return jnp.concatenate([o0, o1], axis=0)
